# baseline (speedup 1.0000x reference)
import jax
import jax.numpy as jnp
from jax import lax
from jax.experimental import pallas as pl
from jax.experimental.pallas import tpu as pltpu
from jax.experimental.pallas import tpu_sc as plsc

_NUM_CORES = 2
_NUM_SUBCORES = 16
_NUM_WORKERS = _NUM_CORES * _NUM_SUBCORES
_LANES = 16
_TC_PER_WAVE = 5
_LIST_CAP = 1024
_WAVE_CAP = 80
_ROW_RING = 16


def _make_body(hidden, table_rows, batch):
    jrs = hidden // 8
    tile_cols = (table_rows + 127) // 128
    n_waves_half = 25
    tcs_per_worker = 2 * n_waves_half * _TC_PER_WAVE
    wave_lanes = _TC_PER_WAVE * 128
    stride_tcs = (tile_cols + _NUM_WORKERS - 1) // _NUM_WORKERS
    n_label_vecs = batch // _LANES

    def body(table3_hbm, idx_hbm, out_hbm, labels_v, rr_v, ii_v, wr_v, wi_v,
             *rest):
        bufs = rest[:2]
        rows = rest[2 : 2 + _ROW_RING]
        wsems = rest[2 + _ROW_RING : 4 + _ROW_RING]
        rsems = rest[4 + _ROW_RING : 4 + 2 * _ROW_RING]
        wid = lax.axis_index("s") * _NUM_CORES + lax.axis_index("c")
        tc0 = jnp.minimum(wid * stride_tcs, tile_cols - tcs_per_worker)
        lane_lo = tc0 * 128
        lane_hi = lane_lo + tcs_per_worker * 128

        pltpu.sync_copy(idx_hbm, labels_v)

        iota = lax.iota(jnp.int32, _LANES)
        lo_v = jnp.full((_LANES,), lane_lo, jnp.int32)
        hi_v = jnp.full((_LANES,), lane_hi, jnp.int32)

        def scan_all(v, off):
            r16 = labels_v[pl.ds(v * _LANES, _LANES)]
            m = (r16 >= lo_v) & (r16 < hi_v)
            plsc.store_compressed(rr_v.at[pl.ds(off, _LANES)], r16, mask=m)
            plsc.store_compressed(
                ii_v.at[pl.ds(off, _LANES)], v * _LANES + iota, mask=m
            )
            return off + plsc.all_reduce_population_count(m)[0]

        acnt = lax.fori_loop(0, n_label_vecs, scan_all, 0)
        acnt_v = jnp.full((_LANES,), acnt, jnp.int32)
        a_vecs = (acnt + _LANES - 1) // _LANES

        gidx = [((jb * _LANES + iota) >> 3, (jb * _LANES + iota) & 7)
                for jb in range(hidden // _LANES)]

        def fire_wave(t, par):
            l0 = pl.multiple_of(lane_lo + t * wave_lanes, 128)
            for r in range(jrs):
                pltpu.async_copy(
                    table3_hbm.at[pl.ds(r, 1), :, pl.ds(l0, wave_lanes)],
                    bufs[par].at[pl.ds(r, 1)],
                    wsems[par],
                )

        def wait_wave(par):
            pltpu.make_async_copy(
                table3_hbm.at[:, :, pl.ds(0, wave_lanes)], bufs[par],
                wsems[par],
            ).wait()

        def process_wave(t, par, fired0):
            wl0 = lane_lo + t * wave_lanes
            wl0_v = jnp.full((_LANES,), wl0, jnp.int32)
            whi_v = wl0_v + wave_lanes

            def rescan(v, wcnt):
                r16 = rr_v[pl.ds(v * _LANES, _LANES)]
                i16 = ii_v[pl.ds(v * _LANES, _LANES)]
                m = ((r16 >= wl0_v) & (r16 < whi_v)
                     & (v * _LANES + iota < acnt_v))
                plsc.store_compressed(wr_v.at[pl.ds(wcnt, _LANES)], r16, mask=m)
                plsc.store_compressed(wi_v.at[pl.ds(wcnt, _LANES)], i16, mask=m)
                return wcnt + plsc.all_reduce_population_count(m)[0]

            wcnt = lax.fori_loop(0, a_vecs, rescan, 0)

            def subgroup(s, fired):
                wr16 = wr_v[pl.ds(s * _LANES, _LANES)]
                wi16 = wi_v[pl.ds(s * _LANES, _LANES)]
                act = (s * _LANES + iota) < jnp.full(
                    (_LANES,), wcnt, jnp.int32
                )
                for j in range(_LANES):
                    @pl.when(s * _LANES + j < wcnt)
                    def _():
                        @pl.when(fired[j] > 0)
                        def _():
                            pltpu.make_async_copy(
                                rows[j], out_hbm.at[pl.ds(0, 1), :],
                                rsems[j],
                            ).wait()

                        lane = jnp.full((_LANES,), wr16[j] - wl0, jnp.int32)
                        zero = jnp.zeros((_LANES,), jnp.int32)
                        for jb, (i0, i1) in enumerate(gidx):
                            val = plsc.load_gather(bufs[par], [i0, i1, lane])
                            plsc.store_scatter(
                                rows[j], [zero, jb * _LANES + iota], val
                            )
                        pltpu.async_copy(
                            rows[j],
                            out_hbm.at[pl.ds(wi16[j], 1), :],
                            rsems[j],
                        )

                return fired | act.astype(jnp.int32)

            nsub = (wcnt + _LANES - 1) // _LANES
            return lax.fori_loop(0, nsub, subgroup, fired0)

        fire_wave(0, 0)

        def pair(p, fired):
            fire_wave(2 * p + 1, 1)
            wait_wave(0)
            fired = process_wave(2 * p, 0, fired)

            @pl.when(p < n_waves_half - 1)
            def _():
                fire_wave(2 * p + 2, 0)

            wait_wave(1)
            return process_wave(2 * p + 1, 1, fired)

        fired = lax.fori_loop(
            0, n_waves_half, pair, jnp.zeros((_LANES,), jnp.int32)
        )
        for s in range(_ROW_RING):
            @pl.when(fired[s] > 0)
            def _():
                pltpu.make_async_copy(
                    rows[s], out_hbm.at[pl.ds(0, 1), :], rsems[s]
                ).wait()

    return body


def kernel(labels, embedding_table):
    batch = labels.shape[0]
    table_rows, hidden = embedding_table.shape
    mesh = plsc.VectorSubcoreMesh(core_axis_name="c", subcore_axis_name="s")
    scratch = [
        pltpu.VMEM((batch,), jnp.int32),
        pltpu.VMEM((_LIST_CAP + _LANES,), jnp.int32),
        pltpu.VMEM((_LIST_CAP + _LANES,), jnp.int32),
        pltpu.VMEM((_WAVE_CAP + _LANES,), jnp.int32),
        pltpu.VMEM((_WAVE_CAP + _LANES,), jnp.int32),
        pltpu.VMEM((hidden // 8, 8, _TC_PER_WAVE * 128), jnp.float32),
        pltpu.VMEM((hidden // 8, 8, _TC_PER_WAVE * 128), jnp.float32),
    ]
    scratch += [pltpu.VMEM((1, hidden), jnp.float32) for _ in range(_ROW_RING)]
    scratch += [pltpu.SemaphoreType.DMA for _ in range(2)]
    scratch += [pltpu.SemaphoreType.DMA for _ in range(_ROW_RING)]
    f = pl.kernel(
        _make_body(hidden, table_rows, batch),
        mesh=mesh,
        out_type=jax.ShapeDtypeStruct((batch, hidden), jnp.float32),
        scratch_types=scratch,
        compiler_params=pltpu.CompilerParams(needs_layout_passes=False),
    )
    table3 = embedding_table.T.reshape(hidden // 8, 8, table_rows)
    return f(table3, labels.astype(jnp.int32))


# speedup vs baseline: 1.0269x; 1.0269x over previous
import jax
import jax.numpy as jnp
from jax import lax
from jax.experimental import pallas as pl
from jax.experimental.pallas import tpu as pltpu
from jax.experimental.pallas import tpu_sc as plsc

_NUM_CORES = 2
_NUM_SUBCORES = 16
_NUM_WORKERS = _NUM_CORES * _NUM_SUBCORES
_LANES = 16
_TC_PER_WAVE = 5
_LIST_CAP = 1024
_WAVE_CAP = 80
_ROW_RING = 16


def _make_body(hidden, table_rows, batch):
    jrs = hidden // 8
    tile_cols = (table_rows + 127) // 128
    n_waves_half = 25
    tcs_per_worker = 2 * n_waves_half * _TC_PER_WAVE
    wave_lanes = _TC_PER_WAVE * 128
    stride_tcs = (tile_cols + _NUM_WORKERS - 1) // _NUM_WORKERS
    n_label_vecs = batch // _LANES

    def body(table3_hbm, idx_hbm, out_hbm, labels_v, rr_v, ii_v, wr_v, wi_v,
             *rest):
        bufs = rest[:2]
        rows = rest[2 : 2 + _ROW_RING]
        wsems = rest[2 + _ROW_RING : 4 + _ROW_RING]
        rsems = rest[4 + _ROW_RING : 4 + 2 * _ROW_RING]
        wid = lax.axis_index("s") * _NUM_CORES + lax.axis_index("c")
        tc0 = jnp.minimum(wid * stride_tcs, tile_cols - tcs_per_worker)
        lane_lo = tc0 * 128
        lane_hi = lane_lo + tcs_per_worker * 128

        pltpu.sync_copy(idx_hbm, labels_v)

        iota = lax.iota(jnp.int32, _LANES)
        lo_v = jnp.full((_LANES,), lane_lo, jnp.int32)
        hi_v = jnp.full((_LANES,), lane_hi, jnp.int32)

        def fire_wave(t, par):
            l0 = pl.multiple_of(lane_lo + t * wave_lanes, 128)
            pltpu.async_copy(
                table3_hbm.at[:, :, pl.ds(l0, wave_lanes)], bufs[par],
                wsems[par],
            )

        fire_wave(0, 0)
        fire_wave(1, 1)

        def scan_all(v, off):
            r16 = labels_v[pl.ds(v * _LANES, _LANES)]
            m = (r16 >= lo_v) & (r16 < hi_v)
            plsc.store_compressed(rr_v.at[pl.ds(off, _LANES)], r16, mask=m)
            plsc.store_compressed(
                ii_v.at[pl.ds(off, _LANES)], v * _LANES + iota, mask=m
            )
            return off + plsc.all_reduce_population_count(m)[0]

        acnt = lax.fori_loop(0, n_label_vecs, scan_all, 0)
        acnt_v = jnp.full((_LANES,), acnt, jnp.int32)
        a_vecs = (acnt + _LANES - 1) // _LANES

        gidx = [((jb * _LANES + iota) >> 3, (jb * _LANES + iota) & 7)
                for jb in range(hidden // _LANES)]

        def wait_wave(par):
            pltpu.make_async_copy(
                table3_hbm.at[:, :, pl.ds(0, wave_lanes)], bufs[par],
                wsems[par],
            ).wait()

        def process_wave(t, par, fired0):
            wl0 = lane_lo + t * wave_lanes
            wl0_v = jnp.full((_LANES,), wl0, jnp.int32)
            whi_v = wl0_v + wave_lanes

            def rescan(v, wcnt):
                r16 = rr_v[pl.ds(v * _LANES, _LANES)]
                i16 = ii_v[pl.ds(v * _LANES, _LANES)]
                m = ((r16 >= wl0_v) & (r16 < whi_v)
                     & (v * _LANES + iota < acnt_v))
                plsc.store_compressed(wr_v.at[pl.ds(wcnt, _LANES)], r16, mask=m)
                plsc.store_compressed(wi_v.at[pl.ds(wcnt, _LANES)], i16, mask=m)
                return wcnt + plsc.all_reduce_population_count(m)[0]

            wcnt = lax.fori_loop(0, a_vecs, rescan, 0)

            def subgroup(s, fired):
                wr16 = wr_v[pl.ds(s * _LANES, _LANES)]
                wi16 = wi_v[pl.ds(s * _LANES, _LANES)]
                act = (s * _LANES + iota) < jnp.full(
                    (_LANES,), wcnt, jnp.int32
                )
                for j in range(_LANES):
                    @pl.when(s * _LANES + j < wcnt)
                    def _():
                        @pl.when(fired[j] > 0)
                        def _():
                            pltpu.make_async_copy(
                                rows[j], out_hbm.at[pl.ds(0, 1), :],
                                rsems[j],
                            ).wait()

                        lane = jnp.full((_LANES,), wr16[j] - wl0, jnp.int32)
                        zero = jnp.zeros((_LANES,), jnp.int32)
                        for jb, (i0, i1) in enumerate(gidx):
                            val = plsc.load_gather(bufs[par], [i0, i1, lane])
                            plsc.store_scatter(
                                rows[j], [zero, jb * _LANES + iota], val
                            )
                        pltpu.async_copy(
                            rows[j],
                            out_hbm.at[pl.ds(wi16[j], 1), :],
                            rsems[j],
                        )

                return fired | act.astype(jnp.int32)

            nsub = (wcnt + _LANES - 1) // _LANES
            return lax.fori_loop(0, nsub, subgroup, fired0)

        def pair(p, fired):
            wait_wave(0)
            fired = process_wave(2 * p, 0, fired)

            @pl.when(p < n_waves_half - 1)
            def _():
                fire_wave(2 * p + 2, 0)

            wait_wave(1)
            fired = process_wave(2 * p + 1, 1, fired)

            @pl.when(p < n_waves_half - 1)
            def _():
                fire_wave(2 * p + 3, 1)

            return fired

        fired = lax.fori_loop(
            0, n_waves_half, pair, jnp.zeros((_LANES,), jnp.int32)
        )
        for s in range(_ROW_RING):
            @pl.when(fired[s] > 0)
            def _():
                pltpu.make_async_copy(
                    rows[s], out_hbm.at[pl.ds(0, 1), :], rsems[s]
                ).wait()

    return body


def kernel(labels, embedding_table):
    batch = labels.shape[0]
    table_rows, hidden = embedding_table.shape
    mesh = plsc.VectorSubcoreMesh(core_axis_name="c", subcore_axis_name="s")
    scratch = [
        pltpu.VMEM((batch,), jnp.int32),
        pltpu.VMEM((_LIST_CAP + _LANES,), jnp.int32),
        pltpu.VMEM((_LIST_CAP + _LANES,), jnp.int32),
        pltpu.VMEM((_WAVE_CAP + _LANES,), jnp.int32),
        pltpu.VMEM((_WAVE_CAP + _LANES,), jnp.int32),
        pltpu.VMEM((hidden // 8, 8, _TC_PER_WAVE * 128), jnp.float32),
        pltpu.VMEM((hidden // 8, 8, _TC_PER_WAVE * 128), jnp.float32),
    ]
    scratch += [pltpu.VMEM((1, hidden), jnp.float32) for _ in range(_ROW_RING)]
    scratch += [pltpu.SemaphoreType.DMA for _ in range(2)]
    scratch += [pltpu.SemaphoreType.DMA for _ in range(_ROW_RING)]
    f = pl.kernel(
        _make_body(hidden, table_rows, batch),
        mesh=mesh,
        out_type=jax.ShapeDtypeStruct((batch, hidden), jnp.float32),
        scratch_types=scratch,
        compiler_params=pltpu.CompilerParams(needs_layout_passes=False),
    )
    table3 = embedding_table.T.reshape(hidden // 8, 8, table_rows)
    return f(table3, labels.astype(jnp.int32))
